# baseline (speedup 1.0000x reference)
import functools

import jax
import jax.numpy as jnp
from jax import lax
from jax.experimental import pallas as pl
from jax.experimental.pallas import tpu as pltpu
from jax.experimental.pallas import tpu_sc as plsc

_D = 64
_WIDE = 128
_LPAD = 56
_NC = 2
_NS = 16
_NW = _NC * _NS
_NBUF = 4
_LANES = 16


def _make_lookup(n_batch, seq_len):
    assert n_batch % (_NW * _NBUF) == 0
    b_per_w = n_batch // _NW
    n_iters = b_per_w // _NBUF
    mesh = plsc.VectorSubcoreMesh(core_axis_name="c", subcore_axis_name="s")

    @functools.partial(
        pl.kernel,
        mesh=mesh,
        out_type=jax.ShapeDtypeStruct((n_batch, seq_len, _D), jnp.float32),
        scratch_types=[pltpu.VMEM((b_per_w * _LPAD + _LANES,), jnp.int32)]
        + [pltpu.VMEM((b_per_w * _LPAD,), jnp.int32)]
        + [pltpu.VMEM((seq_len, _WIDE), jnp.float32) for _ in range(_NBUF)]
        + [pltpu.VMEM((seq_len, _D), jnp.float32) for _ in range(_NBUF)]
        + [pltpu.SemaphoreType.DMA] * (2 * _NBUF),
    )
    def lookup(idx_hbm, table_hbm, out_hbm, idx_v, midx_v, *refs):
        rows = refs[:_NBUF]
        sel = refs[_NBUF:2 * _NBUF]
        gsem = refs[2 * _NBUF:3 * _NBUF]
        wsem = refs[3 * _NBUF:4 * _NBUF]
        wid = lax.axis_index("s") * _NC + lax.axis_index("c")
        batch_base = wid * b_per_w
        pltpu.sync_copy(
            idx_hbm.at[pl.ds(batch_base * _LPAD, b_per_w * _LPAD)],
            idx_v.at[pl.ds(0, b_per_w * _LPAD)])

        def half_idx(i, carry):
            midx_v[pl.ds(i * _LANES, _LANES)] = lax.shift_right_logical(
                idx_v[pl.ds(i * _LANES, _LANES)], 1)
            return carry
        lax.fori_loop(0, b_per_w * _LPAD // _LANES, half_idx, 0)

        def fire_gather(bb, k):
            pltpu.async_copy(
                table_hbm.at[midx_v.at[pl.ds(bb * _LPAD, seq_len)]],
                rows[k],
                gsem[k],
            )

        def drain_gather(k):
            pltpu.make_async_copy(
                table_hbm.at[midx_v.at[pl.ds(0, seq_len)]],
                rows[k],
                gsem[k],
            ).wait()

        def compact(bb, k):
            def per_row(j, carry):
                tok = idx_v[pl.ds(bb * _LPAD + j, _LANES)][0]
                off = (tok & 1) * _D
                for q in range(_D // _LANES):
                    sel[k][j, pl.ds(q * _LANES, _LANES)] = (
                        rows[k][j, pl.ds(off + q * _LANES, _LANES)])
                return carry
            lax.fori_loop(0, seq_len, per_row, 0)

        def fire_write(bb, k):
            pltpu.async_copy(sel[k], out_hbm.at[batch_base + bb], wsem[k])

        def drain_write(k):
            pltpu.make_async_copy(
                sel[k], out_hbm.at[batch_base], wsem[k]).wait()

        for k in range(_NBUF):
            fire_gather(k, k)

        def body(t, carry):
            b0 = t * _NBUF
            for k in range(_NBUF):
                drain_gather(k)

                @pl.when(t > 0)
                def _():
                    drain_write(k)

                compact(b0 + k, k)
                fire_gather(lax.rem(b0 + k + _NBUF, b_per_w), k)
                fire_write(b0 + k, k)
            return carry

        lax.fori_loop(0, n_iters, body, 0)
        for k in range(_NBUF):
            drain_gather(k)
            drain_write(k)

    return lookup


def kernel(token_ids, embeddings):
    from jax.experimental import layout as jlayout
    b, l = token_ids.shape
    v, d = embeddings.shape
    table_pairs = jlayout.with_layout_constraint(
        embeddings.reshape(v // 2, 2 * d),
        jlayout.Layout((0, 1), ((8, 128),)),
    )
    idx_flat = jnp.pad(
        token_ids.astype(jnp.int32), ((0, 0), (0, _LPAD - l))).reshape(-1)
    return _make_lookup(b, l)(idx_flat, table_pairs)


# speedup vs baseline: 1.4375x; 1.4375x over previous
import functools

import jax
import jax.numpy as jnp
from jax import lax
from jax.experimental import pallas as pl
from jax.experimental.pallas import tpu as pltpu
from jax.experimental.pallas import tpu_sc as plsc

_D = 64
_WIDE = 128
_LPAD = 56
_NC = 2
_NS = 16
_NW = _NC * _NS
_NBUF = 4
_LANES = 16


def _make_lookup(n_batch, seq_len):
    assert n_batch % (_NW * _NBUF) == 0
    b_per_w = n_batch // _NW
    n_iters = b_per_w // _NBUF
    mesh = plsc.VectorSubcoreMesh(core_axis_name="c", subcore_axis_name="s")

    @functools.partial(
        pl.kernel,
        mesh=mesh,
        out_type=jax.ShapeDtypeStruct((n_batch, seq_len, _D), jnp.float32),
        scratch_types=[pltpu.VMEM((b_per_w * _LPAD,), jnp.int32)]
        + [pltpu.VMEM((_LPAD, _WIDE), jnp.float32) for _ in range(_NBUF)]
        + [pltpu.VMEM((seq_len, _D), jnp.float32) for _ in range(_NBUF)]
        + [pltpu.SemaphoreType.DMA] * (2 * _NBUF),
    )
    def lookup(idx_hbm, table_hbm, out_hbm, idx_v, *refs):
        rows = refs[:_NBUF]
        sel = refs[_NBUF:2 * _NBUF]
        gsem = refs[2 * _NBUF:3 * _NBUF]
        wsem = refs[3 * _NBUF:4 * _NBUF]
        wid = lax.axis_index("s") * _NC + lax.axis_index("c")
        batch_base = wid * b_per_w
        pltpu.sync_copy(
            idx_hbm.at[pl.ds(batch_base * _LPAD, b_per_w * _LPAD)], idx_v)

        def fire_gather(bb, k):
            pltpu.async_copy(
                table_hbm.at[idx_v.at[pl.ds(bb * _LPAD, seq_len)]],
                rows[k].at[pl.ds(0, seq_len)],
                gsem[k],
            )

        def drain_gather(k):
            pltpu.make_async_copy(
                table_hbm.at[idx_v.at[pl.ds(0, seq_len)]],
                rows[k].at[pl.ds(0, seq_len)],
                gsem[k],
            ).wait()

        def compact(k):
            def per_row(j, carry):
                for q in range(_D // _LANES):
                    sel[k][j, pl.ds(q * _LANES, _LANES)] = (
                        rows[k][j, pl.ds(q * _LANES, _LANES)])
                return carry
            lax.fori_loop(0, seq_len, per_row, 0)

        def fire_write(bb, k):
            pltpu.async_copy(sel[k], out_hbm.at[batch_base + bb], wsem[k])

        def drain_write(k):
            pltpu.make_async_copy(
                sel[k], out_hbm.at[batch_base], wsem[k]).wait()

        for k in range(_NBUF):
            fire_gather(k, k)

        def body(t, carry):
            b0 = t * _NBUF
            for k in range(_NBUF):
                drain_gather(k)

                @pl.when(t > 0)
                def _():
                    drain_write(k)

                compact(k)
                fire_gather(lax.rem(b0 + k + _NBUF, b_per_w), k)
                fire_write(b0 + k, k)
            return carry

        lax.fori_loop(0, n_iters, body, 0)
        for k in range(_NBUF):
            drain_gather(k)
            drain_write(k)

    return lookup


def kernel(token_ids, embeddings):
    from jax.experimental import layout as jlayout
    b, l = token_ids.shape
    v, d = embeddings.shape
    table_wide = jlayout.with_layout_constraint(
        jnp.pad(embeddings, ((0, 0), (0, _WIDE - d))),
        jlayout.Layout((0, 1), ((8, 128),)),
    )
    idx_flat = jnp.pad(
        token_ids.astype(jnp.int32), ((0, 0), (0, _LPAD - l))).reshape(-1)
    return _make_lookup(b, l)(idx_flat, table_wide)
